# Initial kernel scaffold; baseline (speedup 1.0000x reference)
import jax
import jax.numpy as jnp
from jax.experimental import pallas as pl


def kernel(x, edge_index, W1, b1, W2, b2, Wl1, bl1, Wl2, bl2):
    raise NotImplementedError("write your pallas kernel here")


# speedup vs baseline: 34.4958x; 34.4958x over previous
import functools

import jax
import jax.numpy as jnp
from jax import lax
from jax.experimental import pallas as pl
from jax.experimental.pallas import tpu as pltpu
from jax.experimental.pallas import tpu_sc as plsc

NC = 2
NS = 16


def _deg_call(dst, n_pad):
    e = dst.shape[0]
    epw = e // (NC * NS)
    chunk = 1000
    nchunk = epw // chunk
    rpt = n_pad // NS
    ones_buf = 1008

    mesh = plsc.VectorSubcoreMesh(core_axis_name="c", subcore_axis_name="s",
                                  num_cores=NC, num_subcores=NS)

    @functools.partial(
        pl.kernel,
        out_type=[jax.ShapeDtypeStruct((n_pad,), jnp.float32),
                  jax.ShapeDtypeStruct((n_pad,), jnp.float32)],
        mesh=mesh,
        compiler_params=pltpu.CompilerParams(use_tc_tiling_on_sc=False),
        scratch_types=[
            pltpu.VMEM((chunk,), jnp.int32),
            pltpu.VMEM((ones_buf,), jnp.float32),
            pltpu.VMEM(((rpt + 15) // 16 * 16,), jnp.float32),
            pltpu.VMEM_SHARED((n_pad,), jnp.float32),
        ],
    )
    def degk(dst_hbm, out0_hbm, out1_hbm, dst_v, ones_v, stage_v, deg_sh):
        c = lax.axis_index("c")
        s = lax.axis_index("s")
        base = (s * NC + c) * epw

        def fill(i, _):
            ones_v[pl.ds(i * 16, 16)] = jnp.ones((16,), jnp.float32)
            return 0

        lax.fori_loop(0, ones_buf // 16, fill, 0)

        def fillz(i, _):
            stage_v[pl.ds(i * 16, 16)] = jnp.zeros((16,), jnp.float32)
            return 0

        lax.fori_loop(0, stage_v.shape[0] // 16, fillz, 0)
        pltpu.sync_copy(stage_v.at[pl.ds(0, rpt)],
                        deg_sh.at[pl.ds(s * rpt, rpt)])
        plsc.subcore_barrier()

        def body(k, _):
            off = pl.multiple_of(base + k * chunk, 8)
            pltpu.sync_copy(dst_hbm.at[pl.ds(off, chunk)], dst_v)
            pltpu.sync_copy(ones_v.at[pl.ds(0, chunk)],
                            deg_sh.at[dst_v], add=True)
            return 0

        lax.fori_loop(0, nchunk, body, 0)
        plsc.subcore_barrier()
        pltpu.sync_copy(deg_sh.at[pl.ds(s * rpt, rpt)],
                        stage_v.at[pl.ds(0, rpt)])

        @pl.when(c == 0)
        def _():
            pltpu.sync_copy(stage_v.at[pl.ds(0, rpt)],
                            out0_hbm.at[pl.ds(s * rpt, rpt)])

        @pl.when(c == 1)
        def _():
            pltpu.sync_copy(stage_v.at[pl.ds(0, rpt)],
                            out1_hbm.at[pl.ds(s * rpt, rpt)])

    return degk(dst)


def _msgpass_call(y, src, dst, zeros2):
    n, h = y.shape
    n_pad = zeros2.shape[0]
    e = src.shape[0]
    epw = e // (NC * NS)
    chunk = 1000
    nchunk = epw // chunk
    rpt = n_pad // NS

    mesh = plsc.VectorSubcoreMesh(core_axis_name="c", subcore_axis_name="s",
                                  num_cores=NC, num_subcores=NS)

    @functools.partial(
        pl.kernel,
        out_type=[jax.ShapeDtypeStruct((n_pad, h), jnp.float32),
                  jax.ShapeDtypeStruct((n_pad, h), jnp.float32)],
        mesh=mesh,
        compiler_params=pltpu.CompilerParams(use_tc_tiling_on_sc=False),
        scratch_types=[
            pltpu.VMEM((chunk,), jnp.int32),
            pltpu.VMEM((chunk,), jnp.int32),
            pltpu.VMEM((chunk, h), jnp.float32),
            pltpu.VMEM_SHARED((n_pad, h), jnp.float32),
            pltpu.SemaphoreType.DMA,
        ],
    )
    def msgk(y_hbm, src_hbm, dst_hbm, z_hbm, out0_hbm, out1_hbm,
             src_v, dst_v, rows_v, acc_sh, sem):
        c = lax.axis_index("c")
        s = lax.axis_index("s")
        base = (s * NC + c) * epw

        pltpu.sync_copy(z_hbm.at[pl.ds(s * rpt, rpt)],
                        rows_v.at[pl.ds(0, rpt)])
        pltpu.sync_copy(rows_v.at[pl.ds(0, rpt)],
                        acc_sh.at[pl.ds(s * rpt, rpt)])
        plsc.subcore_barrier()

        def body(k, _):
            off = pl.multiple_of(base + k * chunk, 8)
            pltpu.sync_copy(src_hbm.at[pl.ds(off, chunk)], src_v)
            pltpu.sync_copy(dst_hbm.at[pl.ds(off, chunk)], dst_v)
            pltpu.async_copy(y_hbm.at[src_v], rows_v, sem).wait()
            pltpu.sync_copy(rows_v, acc_sh.at[dst_v], add=True)
            return 0

        lax.fori_loop(0, nchunk, body, 0)
        plsc.subcore_barrier()
        pltpu.sync_copy(acc_sh.at[pl.ds(s * rpt, rpt)],
                        rows_v.at[pl.ds(0, rpt)])

        @pl.when(c == 0)
        def _():
            pltpu.sync_copy(rows_v.at[pl.ds(0, rpt)],
                            out0_hbm.at[pl.ds(s * rpt, rpt)])

        @pl.when(c == 1)
        def _():
            pltpu.sync_copy(rows_v.at[pl.ds(0, rpt)],
                            out1_hbm.at[pl.ds(s * rpt, rpt)])

    return msgk(y, src, dst, zeros2)


def _tc_stage1(x, W1, deg0, deg1):
    n, d_in = x.shape
    h1 = W1.shape[1]
    bm = 2000
    g = n // bm

    def body(x_ref, w_ref, d0_ref, d1_ref, y_ref, dis_ref):
        dis = lax.rsqrt(d0_ref[...] + d1_ref[...] + 1.0)
        xw = jnp.dot(x_ref[...], w_ref[...],
                     preferred_element_type=jnp.float32)
        y_ref[...] = xw * dis
        dis_ref[...] = dis

    return pl.pallas_call(
        body,
        grid=(g,),
        in_specs=[
            pl.BlockSpec((bm, d_in), lambda i: (i, 0)),
            pl.BlockSpec((d_in, h1), lambda i: (0, 0)),
            pl.BlockSpec((bm, 1), lambda i: (i, 0)),
            pl.BlockSpec((bm, 1), lambda i: (i, 0)),
        ],
        out_specs=[
            pl.BlockSpec((bm, h1), lambda i: (i, 0)),
            pl.BlockSpec((bm, 1), lambda i: (i, 0)),
        ],
        out_shape=[
            jax.ShapeDtypeStruct((n, h1), jnp.float32),
            jax.ShapeDtypeStruct((n, 1), jnp.float32),
        ],
    )(x, W1, deg0, deg1)


def _tc_stage2(p0, p1, y1, dis, b1, W2):
    n, h1 = y1.shape
    h2 = W2.shape[1]
    bm = 2000
    g = n // bm

    def body(p0_ref, p1_ref, y_ref, dis_ref, b_ref, w_ref, y2_ref):
        dis = dis_ref[...]
        hh = dis * (p0_ref[...] + p1_ref[...] + y_ref[...]) + b_ref[...]
        hh = jnp.maximum(hh, 0.0)
        y2_ref[...] = dis * jnp.dot(hh, w_ref[...],
                                    preferred_element_type=jnp.float32)

    return pl.pallas_call(
        body,
        grid=(g,),
        in_specs=[
            pl.BlockSpec((bm, h1), lambda i: (i, 0)),
            pl.BlockSpec((bm, h1), lambda i: (i, 0)),
            pl.BlockSpec((bm, h1), lambda i: (i, 0)),
            pl.BlockSpec((bm, 1), lambda i: (i, 0)),
            pl.BlockSpec((1, h1), lambda i: (0, 0)),
            pl.BlockSpec((h1, h2), lambda i: (0, 0)),
        ],
        out_specs=pl.BlockSpec((bm, h2), lambda i: (i, 0)),
        out_shape=jax.ShapeDtypeStruct((n, h2), jnp.float32),
    )(p0, p1, y1, dis, b1, W2)


def _tc_stage3(q0, q1, y2, dis, b2, Wl1, bl1, Wl2, bl2):
    n, h2 = y2.shape
    l2 = Wl2.shape[1]

    def body(q0_ref, q1_ref, y_ref, dis_ref, b2_ref, w1_ref, c1_ref,
             w2_ref, c2_ref, out_ref):
        hh = dis_ref[...] * (q0_ref[...] + q1_ref[...] + y_ref[...]) \
            + b2_ref[...]
        t = jnp.dot(hh, w1_ref[...],
                    preferred_element_type=jnp.float32) + c1_ref[...]
        lg = jnp.dot(t, w2_ref[...],
                     preferred_element_type=jnp.float32) + c2_ref[...]
        m = jnp.max(lg, axis=0, keepdims=True)
        ex = jnp.exp(lg - m)
        sm = jnp.sum(ex, axis=0, keepdims=True)
        out_ref[...] = ex / sm

    h1 = q0.shape[1]
    l1 = Wl1.shape[1]
    return pl.pallas_call(
        body,
        grid=(1,),
        in_specs=[
            pl.BlockSpec((n, h1), lambda i: (0, 0)),
            pl.BlockSpec((n, h1), lambda i: (0, 0)),
            pl.BlockSpec((n, h2), lambda i: (0, 0)),
            pl.BlockSpec((n, 1), lambda i: (0, 0)),
            pl.BlockSpec((1, h2), lambda i: (0, 0)),
            pl.BlockSpec((h2, l1), lambda i: (0, 0)),
            pl.BlockSpec((1, l1), lambda i: (0, 0)),
            pl.BlockSpec((l1, l2), lambda i: (0, 0)),
            pl.BlockSpec((1, l2), lambda i: (0, 0)),
        ],
        out_specs=pl.BlockSpec((n, l2), lambda i: (0, 0)),
        out_shape=jax.ShapeDtypeStruct((n, l2), jnp.float32),
    )(q0, q1, y2, dis, b2, Wl1, bl1, Wl2, bl2)


def kernel(x, edge_index, W1, b1, W2, b2, Wl1, bl1, Wl2, bl2):
    n = x.shape[0]
    h1 = W1.shape[1]
    n_pad = ((n + 16 * 8 - 1) // (16 * 8)) * (16 * 8)

    ei = edge_index.astype(jnp.int32)
    src, dst = ei[0], ei[1]

    deg0, deg1 = _deg_call(dst, n_pad)
    deg0 = deg0.reshape(n_pad, 1)
    deg1 = deg1.reshape(n_pad, 1)

    y1, dis = _tc_stage1(x, W1, deg0, deg1)

    zeros2 = jnp.zeros((n_pad, h1), jnp.float32)
    p0, p1 = _msgpass_call(y1, src, dst, zeros2)
    y2 = _tc_stage2(p0, p1, y1, dis, b1.reshape(1, -1), W2)

    q0, q1 = _msgpass_call(y2, src, dst, zeros2)
    return _tc_stage3(q0, q1, y2, dis, b2.reshape(1, -1),
                      Wl1, bl1.reshape(1, -1), Wl2, bl2.reshape(1, -1))


# speedup vs baseline: 38.8872x; 1.1273x over previous
import functools

import jax
import jax.numpy as jnp
from jax import lax
from jax.experimental import pallas as pl
from jax.experimental.pallas import tpu as pltpu
from jax.experimental.pallas import tpu_sc as plsc

NC = 2
NS = 16


def _deg_call(dst, n_pad):
    e = dst.shape[0]
    epw = e // (NC * NS)
    chunk = 1000
    nchunk = epw // chunk
    rpt = n_pad // NS
    ones_buf = 1008

    mesh = plsc.VectorSubcoreMesh(core_axis_name="c", subcore_axis_name="s",
                                  num_cores=NC, num_subcores=NS)

    @functools.partial(
        pl.kernel,
        out_type=[jax.ShapeDtypeStruct((n_pad,), jnp.float32),
                  jax.ShapeDtypeStruct((n_pad,), jnp.float32)],
        mesh=mesh,
        compiler_params=pltpu.CompilerParams(use_tc_tiling_on_sc=False),
        scratch_types=[
            pltpu.VMEM((chunk,), jnp.int32),
            pltpu.VMEM((ones_buf,), jnp.float32),
            pltpu.VMEM(((rpt + 15) // 16 * 16,), jnp.float32),
            pltpu.VMEM_SHARED((n_pad,), jnp.float32),
        ],
    )
    def degk(dst_hbm, out0_hbm, out1_hbm, dst_v, ones_v, stage_v, deg_sh):
        c = lax.axis_index("c")
        s = lax.axis_index("s")
        base = (s * NC + c) * epw

        def fill(i, _):
            ones_v[pl.ds(i * 16, 16)] = jnp.ones((16,), jnp.float32)
            return 0

        lax.fori_loop(0, ones_buf // 16, fill, 0)

        def fillz(i, _):
            stage_v[pl.ds(i * 16, 16)] = jnp.zeros((16,), jnp.float32)
            return 0

        lax.fori_loop(0, stage_v.shape[0] // 16, fillz, 0)
        pltpu.sync_copy(stage_v.at[pl.ds(0, rpt)],
                        deg_sh.at[pl.ds(s * rpt, rpt)])
        plsc.subcore_barrier()

        def body(k, _):
            off = pl.multiple_of(base + k * chunk, 8)
            pltpu.sync_copy(dst_hbm.at[pl.ds(off, chunk)], dst_v)
            pltpu.sync_copy(ones_v.at[pl.ds(0, chunk)],
                            deg_sh.at[dst_v], add=True)
            return 0

        lax.fori_loop(0, nchunk, body, 0)
        plsc.subcore_barrier()
        pltpu.sync_copy(deg_sh.at[pl.ds(s * rpt, rpt)],
                        stage_v.at[pl.ds(0, rpt)])

        @pl.when(c == 0)
        def _():
            pltpu.sync_copy(stage_v.at[pl.ds(0, rpt)],
                            out0_hbm.at[pl.ds(s * rpt, rpt)])

        @pl.when(c == 1)
        def _():
            pltpu.sync_copy(stage_v.at[pl.ds(0, rpt)],
                            out1_hbm.at[pl.ds(s * rpt, rpt)])

    return degk(dst)


def _msgpass_call(y, src, dst, zeros2):
    n, h = y.shape
    n_pad = zeros2.shape[0]
    e = src.shape[0]
    epw = e // (NC * NS)
    chunk = 400
    nchunk = epw // chunk
    npair = (nchunk - 1) // 2
    rpt = n_pad // NS

    mesh = plsc.VectorSubcoreMesh(core_axis_name="c", subcore_axis_name="s",
                                  num_cores=NC, num_subcores=NS)

    @functools.partial(
        pl.kernel,
        out_type=[jax.ShapeDtypeStruct((n_pad, h), jnp.float32),
                  jax.ShapeDtypeStruct((n_pad, h), jnp.float32)],
        mesh=mesh,
        compiler_params=pltpu.CompilerParams(use_tc_tiling_on_sc=False),
        scratch_types=[
            pltpu.VMEM((chunk,), jnp.int32),
            pltpu.VMEM((chunk,), jnp.int32),
            pltpu.VMEM((chunk,), jnp.int32),
            pltpu.VMEM((chunk,), jnp.int32),
            pltpu.VMEM((chunk, h), jnp.float32),
            pltpu.VMEM((chunk, h), jnp.float32),
            pltpu.VMEM_SHARED((n_pad, h), jnp.float32),
            pltpu.SemaphoreType.DMA,
            pltpu.SemaphoreType.DMA,
        ],
    )
    def msgk(y_hbm, src_hbm, dst_hbm, z_hbm, out0_hbm, out1_hbm,
             src_a, dst_a, src_b, dst_b, rows_a, rows_b,
             acc_sh, sem_a, sem_b):
        c = lax.axis_index("c")
        s = lax.axis_index("s")
        base = (s * NC + c) * epw
        hrpt = rpt // 2

        for half in range(2):
            r0 = s * rpt + half * hrpt
            pltpu.sync_copy(z_hbm.at[pl.ds(r0, hrpt)],
                            rows_a.at[pl.ds(0, hrpt)])
            pltpu.sync_copy(rows_a.at[pl.ds(0, hrpt)],
                            acc_sh.at[pl.ds(r0, hrpt)])
        plsc.subcore_barrier()

        def load_start(k, idx_s, idx_d, rows, sem):
            off = pl.multiple_of(base + k * chunk, 8)
            pltpu.sync_copy(src_hbm.at[pl.ds(off, chunk)], idx_s)
            pltpu.sync_copy(dst_hbm.at[pl.ds(off, chunk)], idx_d)
            pltpu.async_copy(y_hbm.at[idx_s], rows, sem)

        def wait_scatter(idx_s, idx_d, rows, sem):
            pltpu.make_async_copy(y_hbm.at[idx_s], rows, sem).wait()
            pltpu.sync_copy(rows, acc_sh.at[idx_d], add=True)

        load_start(0, src_a, dst_a, rows_a, sem_a)

        def body(j, _):
            load_start(2 * j + 1, src_b, dst_b, rows_b, sem_b)
            wait_scatter(src_a, dst_a, rows_a, sem_a)
            load_start(2 * j + 2, src_a, dst_a, rows_a, sem_a)
            wait_scatter(src_b, dst_b, rows_b, sem_b)
            return 0

        lax.fori_loop(0, npair, body, 0)
        wait_scatter(src_a, dst_a, rows_a, sem_a)

        plsc.subcore_barrier()
        for half in range(2):
            r0 = s * rpt + half * hrpt
            pltpu.sync_copy(acc_sh.at[pl.ds(r0, hrpt)],
                            rows_a.at[pl.ds(0, hrpt)])

            @pl.when(c == 0)
            def _():
                pltpu.sync_copy(rows_a.at[pl.ds(0, hrpt)],
                                out0_hbm.at[pl.ds(r0, hrpt)])

            @pl.when(c == 1)
            def _():
                pltpu.sync_copy(rows_a.at[pl.ds(0, hrpt)],
                                out1_hbm.at[pl.ds(r0, hrpt)])

    return msgk(y, src, dst, zeros2)


def _tc_stage1(x, W1, deg0, deg1):
    n, d_in = x.shape
    h1 = W1.shape[1]
    bm = 2000
    g = n // bm

    def body(x_ref, w_ref, d0_ref, d1_ref, y_ref, dis_ref):
        dis = lax.rsqrt(d0_ref[...] + d1_ref[...] + 1.0)
        xw = jnp.dot(x_ref[...], w_ref[...],
                     preferred_element_type=jnp.float32)
        y_ref[...] = xw * dis
        dis_ref[...] = dis

    return pl.pallas_call(
        body,
        grid=(g,),
        in_specs=[
            pl.BlockSpec((bm, d_in), lambda i: (i, 0)),
            pl.BlockSpec((d_in, h1), lambda i: (0, 0)),
            pl.BlockSpec((bm, 1), lambda i: (i, 0)),
            pl.BlockSpec((bm, 1), lambda i: (i, 0)),
        ],
        out_specs=[
            pl.BlockSpec((bm, h1), lambda i: (i, 0)),
            pl.BlockSpec((bm, 1), lambda i: (i, 0)),
        ],
        out_shape=[
            jax.ShapeDtypeStruct((n, h1), jnp.float32),
            jax.ShapeDtypeStruct((n, 1), jnp.float32),
        ],
    )(x, W1, deg0, deg1)


def _tc_stage2(p0, p1, y1, dis, b1, W2):
    n, h1 = y1.shape
    h2 = W2.shape[1]
    bm = 2000
    g = n // bm

    def body(p0_ref, p1_ref, y_ref, dis_ref, b_ref, w_ref, y2_ref):
        dis = dis_ref[...]
        hh = dis * (p0_ref[...] + p1_ref[...] + y_ref[...]) + b_ref[...]
        hh = jnp.maximum(hh, 0.0)
        y2_ref[...] = dis * jnp.dot(hh, w_ref[...],
                                    preferred_element_type=jnp.float32)

    return pl.pallas_call(
        body,
        grid=(g,),
        in_specs=[
            pl.BlockSpec((bm, h1), lambda i: (i, 0)),
            pl.BlockSpec((bm, h1), lambda i: (i, 0)),
            pl.BlockSpec((bm, h1), lambda i: (i, 0)),
            pl.BlockSpec((bm, 1), lambda i: (i, 0)),
            pl.BlockSpec((1, h1), lambda i: (0, 0)),
            pl.BlockSpec((h1, h2), lambda i: (0, 0)),
        ],
        out_specs=pl.BlockSpec((bm, h2), lambda i: (i, 0)),
        out_shape=jax.ShapeDtypeStruct((n, h2), jnp.float32),
    )(p0, p1, y1, dis, b1, W2)


def _tc_stage3(q0, q1, y2, dis, b2, Wl1, bl1, Wl2, bl2):
    n, h2 = y2.shape
    l2 = Wl2.shape[1]

    def body(q0_ref, q1_ref, y_ref, dis_ref, b2_ref, w1_ref, c1_ref,
             w2_ref, c2_ref, out_ref):
        hh = dis_ref[...] * (q0_ref[...] + q1_ref[...] + y_ref[...]) \
            + b2_ref[...]
        t = jnp.dot(hh, w1_ref[...],
                    preferred_element_type=jnp.float32) + c1_ref[...]
        lg = jnp.dot(t, w2_ref[...],
                     preferred_element_type=jnp.float32) + c2_ref[...]
        m = jnp.max(lg, axis=0, keepdims=True)
        ex = jnp.exp(lg - m)
        sm = jnp.sum(ex, axis=0, keepdims=True)
        out_ref[...] = ex / sm

    h1 = q0.shape[1]
    l1 = Wl1.shape[1]
    return pl.pallas_call(
        body,
        grid=(1,),
        in_specs=[
            pl.BlockSpec((n, h1), lambda i: (0, 0)),
            pl.BlockSpec((n, h1), lambda i: (0, 0)),
            pl.BlockSpec((n, h2), lambda i: (0, 0)),
            pl.BlockSpec((n, 1), lambda i: (0, 0)),
            pl.BlockSpec((1, h2), lambda i: (0, 0)),
            pl.BlockSpec((h2, l1), lambda i: (0, 0)),
            pl.BlockSpec((1, l1), lambda i: (0, 0)),
            pl.BlockSpec((l1, l2), lambda i: (0, 0)),
            pl.BlockSpec((1, l2), lambda i: (0, 0)),
        ],
        out_specs=pl.BlockSpec((n, l2), lambda i: (0, 0)),
        out_shape=jax.ShapeDtypeStruct((n, l2), jnp.float32),
    )(q0, q1, y2, dis, b2, Wl1, bl1, Wl2, bl2)


def kernel(x, edge_index, W1, b1, W2, b2, Wl1, bl1, Wl2, bl2):
    n = x.shape[0]
    h1 = W1.shape[1]
    n_pad = ((n + 16 * 8 - 1) // (16 * 8)) * (16 * 8)

    ei = edge_index.astype(jnp.int32)
    src, dst = ei[0], ei[1]

    deg0, deg1 = _deg_call(dst, n_pad)
    deg0 = deg0.reshape(n_pad, 1)
    deg1 = deg1.reshape(n_pad, 1)

    y1, dis = _tc_stage1(x, W1, deg0, deg1)

    zeros2 = jnp.zeros((n_pad, h1), jnp.float32)
    p0, p1 = _msgpass_call(y1, src, dst, zeros2)
    y2 = _tc_stage2(p0, p1, y1, dis, b1.reshape(1, -1), W2)

    q0, q1 = _msgpass_call(y2, src, dst, zeros2)
    return _tc_stage3(q0, q1, y2, dis, b2.reshape(1, -1),
                      Wl1, bl1.reshape(1, -1), Wl2, bl2.reshape(1, -1))


# speedup vs baseline: 43.1832x; 1.1105x over previous
import functools

import jax
import jax.numpy as jnp
from jax import lax
from jax.experimental import pallas as pl
from jax.experimental.pallas import tpu as pltpu
from jax.experimental.pallas import tpu_sc as plsc

NC = 2
NS = 16


def _deg_call(dst2, n_pad):
    nrow, chunk = dst2.shape
    e = nrow * chunk
    epw = e // (NC * NS)
    nchunk = epw // chunk
    rpt = n_pad // NS

    mesh = plsc.VectorSubcoreMesh(core_axis_name="c", subcore_axis_name="s",
                                  num_cores=NC, num_subcores=NS)

    @functools.partial(
        pl.kernel,
        out_type=[jax.ShapeDtypeStruct((n_pad,), jnp.float32),
                  jax.ShapeDtypeStruct((n_pad,), jnp.float32)],
        mesh=mesh,
        compiler_params=pltpu.CompilerParams(use_tc_tiling_on_sc=False),
        scratch_types=[
            pltpu.VMEM((nchunk, chunk), jnp.int32),
            pltpu.VMEM((chunk,), jnp.float32),
            pltpu.VMEM(((rpt + 15) // 16 * 16,), jnp.float32),
            pltpu.VMEM_SHARED((n_pad,), jnp.float32),
        ],
    )
    def degk(dst_hbm, out0_hbm, out1_hbm, dst_all, ones_v, stage_v, deg_sh):
        c = lax.axis_index("c")
        s = lax.axis_index("s")
        wid = s * NC + c

        pltpu.sync_copy(dst_hbm.at[pl.ds(wid * nchunk, nchunk), :], dst_all)

        def fill(i, _):
            ones_v[pl.ds(i * 16, 16)] = jnp.ones((16,), jnp.float32)
            return 0

        lax.fori_loop(0, chunk // 16, fill, 0)

        def fillz(i, _):
            stage_v[pl.ds(i * 16, 16)] = jnp.zeros((16,), jnp.float32)
            return 0

        lax.fori_loop(0, stage_v.shape[0] // 16, fillz, 0)
        pltpu.sync_copy(stage_v.at[pl.ds(0, rpt)],
                        deg_sh.at[pl.ds(s * rpt, rpt)])
        plsc.subcore_barrier()

        def body(k, _):
            pltpu.sync_copy(ones_v, deg_sh.at[dst_all.at[k]], add=True)
            return 0

        lax.fori_loop(0, nchunk, body, 0)
        plsc.subcore_barrier()
        pltpu.sync_copy(deg_sh.at[pl.ds(s * rpt, rpt)],
                        stage_v.at[pl.ds(0, rpt)])

        @pl.when(c == 0)
        def _():
            pltpu.sync_copy(stage_v.at[pl.ds(0, rpt)],
                            out0_hbm.at[pl.ds(s * rpt, rpt)])

        @pl.when(c == 1)
        def _():
            pltpu.sync_copy(stage_v.at[pl.ds(0, rpt)],
                            out1_hbm.at[pl.ds(s * rpt, rpt)])

    return degk(dst2)


def _msgpass_call(y, src2, dst2, zeros2):
    n, h = y.shape
    n_pad = zeros2.shape[0]
    nrow, chunk = src2.shape
    e = nrow * chunk
    epw = e // (NC * NS)
    nchunk = epw // chunk
    npair = (nchunk - 1) // 2
    rpt = n_pad // NS

    mesh = plsc.VectorSubcoreMesh(core_axis_name="c", subcore_axis_name="s",
                                  num_cores=NC, num_subcores=NS)

    @functools.partial(
        pl.kernel,
        out_type=[jax.ShapeDtypeStruct((n_pad, h), jnp.float32),
                  jax.ShapeDtypeStruct((n_pad, h), jnp.float32)],
        mesh=mesh,
        compiler_params=pltpu.CompilerParams(use_tc_tiling_on_sc=False),
        scratch_types=[
            pltpu.VMEM((nchunk, chunk), jnp.int32),
            pltpu.VMEM((nchunk, chunk), jnp.int32),
            pltpu.VMEM((chunk, h), jnp.float32),
            pltpu.VMEM((chunk, h), jnp.float32),
            pltpu.VMEM_SHARED((n_pad, h), jnp.float32),
            pltpu.SemaphoreType.DMA,
            pltpu.SemaphoreType.DMA,
        ],
    )
    def msgk(y_hbm, src_hbm, dst_hbm, z_hbm, out0_hbm, out1_hbm,
             src_all, dst_all, rows_a, rows_b,
             acc_sh, sem_a, sem_b):
        c = lax.axis_index("c")
        s = lax.axis_index("s")
        wid = s * NC + c
        hrpt = rpt // 2

        pltpu.sync_copy(src_hbm.at[pl.ds(wid * nchunk, nchunk), :], src_all)
        pltpu.sync_copy(dst_hbm.at[pl.ds(wid * nchunk, nchunk), :], dst_all)
        for half in range(2):
            r0 = s * rpt + half * hrpt
            pltpu.sync_copy(z_hbm.at[pl.ds(r0, hrpt)],
                            rows_a.at[pl.ds(0, hrpt)])
            pltpu.sync_copy(rows_a.at[pl.ds(0, hrpt)],
                            acc_sh.at[pl.ds(r0, hrpt)])
        plsc.subcore_barrier()

        def start(k, rows, sem):
            pltpu.async_copy(y_hbm.at[src_all.at[k]], rows, sem)

        def wait_scatter(k, rows, sem):
            pltpu.make_async_copy(y_hbm.at[src_all.at[k]], rows, sem).wait()
            pltpu.sync_copy(rows, acc_sh.at[dst_all.at[k]], add=True)

        start(0, rows_a, sem_a)

        def body(j, _):
            start(2 * j + 1, rows_b, sem_b)
            wait_scatter(2 * j, rows_a, sem_a)
            start(2 * j + 2, rows_a, sem_a)
            wait_scatter(2 * j + 1, rows_b, sem_b)
            return 0

        lax.fori_loop(0, npair, body, 0)
        wait_scatter(nchunk - 1, rows_a, sem_a)

        plsc.subcore_barrier()
        for half in range(2):
            r0 = s * rpt + half * hrpt
            pltpu.sync_copy(acc_sh.at[pl.ds(r0, hrpt)],
                            rows_a.at[pl.ds(0, hrpt)])

            @pl.when(c == 0)
            def _():
                pltpu.sync_copy(rows_a.at[pl.ds(0, hrpt)],
                                out0_hbm.at[pl.ds(r0, hrpt)])

            @pl.when(c == 1)
            def _():
                pltpu.sync_copy(rows_a.at[pl.ds(0, hrpt)],
                                out1_hbm.at[pl.ds(r0, hrpt)])

    return msgk(y, src2, dst2, zeros2)


def _tc_stage1(x, W1, deg0, deg1):
    n, d_in = x.shape
    h1 = W1.shape[1]
    bm = 2000
    g = n // bm

    def body(x_ref, w_ref, d0_ref, d1_ref, y_ref, dis_ref):
        dis = lax.rsqrt(d0_ref[...] + d1_ref[...] + 1.0)
        xw = jnp.dot(x_ref[...], w_ref[...],
                     preferred_element_type=jnp.float32)
        y_ref[...] = xw * dis
        dis_ref[...] = dis

    return pl.pallas_call(
        body,
        grid=(g,),
        in_specs=[
            pl.BlockSpec((bm, d_in), lambda i: (i, 0)),
            pl.BlockSpec((d_in, h1), lambda i: (0, 0)),
            pl.BlockSpec((bm, 1), lambda i: (i, 0)),
            pl.BlockSpec((bm, 1), lambda i: (i, 0)),
        ],
        out_specs=[
            pl.BlockSpec((bm, h1), lambda i: (i, 0)),
            pl.BlockSpec((bm, 1), lambda i: (i, 0)),
        ],
        out_shape=[
            jax.ShapeDtypeStruct((n, h1), jnp.float32),
            jax.ShapeDtypeStruct((n, 1), jnp.float32),
        ],
    )(x, W1, deg0, deg1)


def _tc_stage2(p0, p1, y1, dis, b1, W2):
    n, h1 = y1.shape
    h2 = W2.shape[1]
    bm = 2000
    g = n // bm

    def body(p0_ref, p1_ref, y_ref, dis_ref, b_ref, w_ref, y2_ref):
        dis = dis_ref[...]
        hh = dis * (p0_ref[...] + p1_ref[...] + y_ref[...]) + b_ref[...]
        hh = jnp.maximum(hh, 0.0)
        y2_ref[...] = dis * jnp.dot(hh, w_ref[...],
                                    preferred_element_type=jnp.float32)

    return pl.pallas_call(
        body,
        grid=(g,),
        in_specs=[
            pl.BlockSpec((bm, h1), lambda i: (i, 0)),
            pl.BlockSpec((bm, h1), lambda i: (i, 0)),
            pl.BlockSpec((bm, h1), lambda i: (i, 0)),
            pl.BlockSpec((bm, 1), lambda i: (i, 0)),
            pl.BlockSpec((1, h1), lambda i: (0, 0)),
            pl.BlockSpec((h1, h2), lambda i: (0, 0)),
        ],
        out_specs=pl.BlockSpec((bm, h2), lambda i: (i, 0)),
        out_shape=jax.ShapeDtypeStruct((n, h2), jnp.float32),
    )(p0, p1, y1, dis, b1, W2)


def _tc_stage3(q0, q1, y2, dis, b2, Wl1, bl1, Wl2, bl2):
    n, h2 = y2.shape
    l2 = Wl2.shape[1]

    def body(q0_ref, q1_ref, y_ref, dis_ref, b2_ref, w1_ref, c1_ref,
             w2_ref, c2_ref, out_ref):
        hh = dis_ref[...] * (q0_ref[...] + q1_ref[...] + y_ref[...]) \
            + b2_ref[...]
        t = jnp.dot(hh, w1_ref[...],
                    preferred_element_type=jnp.float32) + c1_ref[...]
        lg = jnp.dot(t, w2_ref[...],
                     preferred_element_type=jnp.float32) + c2_ref[...]
        m = jnp.max(lg, axis=0, keepdims=True)
        ex = jnp.exp(lg - m)
        sm = jnp.sum(ex, axis=0, keepdims=True)
        out_ref[...] = ex / sm

    h1 = q0.shape[1]
    l1 = Wl1.shape[1]
    return pl.pallas_call(
        body,
        grid=(1,),
        in_specs=[
            pl.BlockSpec((n, h1), lambda i: (0, 0)),
            pl.BlockSpec((n, h1), lambda i: (0, 0)),
            pl.BlockSpec((n, h2), lambda i: (0, 0)),
            pl.BlockSpec((n, 1), lambda i: (0, 0)),
            pl.BlockSpec((1, h2), lambda i: (0, 0)),
            pl.BlockSpec((h2, l1), lambda i: (0, 0)),
            pl.BlockSpec((1, l1), lambda i: (0, 0)),
            pl.BlockSpec((l1, l2), lambda i: (0, 0)),
            pl.BlockSpec((1, l2), lambda i: (0, 0)),
        ],
        out_specs=pl.BlockSpec((n, l2), lambda i: (0, 0)),
        out_shape=jax.ShapeDtypeStruct((n, l2), jnp.float32),
    )(q0, q1, y2, dis, b2, Wl1, bl1, Wl2, bl2)


def kernel(x, edge_index, W1, b1, W2, b2, Wl1, bl1, Wl2, bl2):
    n = x.shape[0]
    h1 = W1.shape[1]
    n_pad = ((n + 16 * 8 - 1) // (16 * 8)) * (16 * 8)

    ei = edge_index.astype(jnp.int32)
    chunk = 400
    src2 = ei[0].reshape(-1, chunk)
    dst2 = ei[1].reshape(-1, chunk)

    deg0, deg1 = _deg_call(dst2, n_pad)
    deg0 = deg0.reshape(n_pad, 1)
    deg1 = deg1.reshape(n_pad, 1)

    y1, dis = _tc_stage1(x, W1, deg0, deg1)

    zeros2 = jnp.zeros((n_pad, h1), jnp.float32)
    p0, p1 = _msgpass_call(y1, src2, dst2, zeros2)
    y2 = _tc_stage2(p0, p1, y1, dis, b1.reshape(1, -1), W2)

    q0, q1 = _msgpass_call(y2, src2, dst2, zeros2)
    return _tc_stage3(q0, q1, y2, dis, b2.reshape(1, -1),
                      Wl1, bl1.reshape(1, -1), Wl2, bl2.reshape(1, -1))


# speedup vs baseline: 48.2700x; 1.1178x over previous
import functools

import jax
import jax.numpy as jnp
from jax import lax
from jax.experimental import pallas as pl
from jax.experimental.pallas import tpu as pltpu
from jax.experimental.pallas import tpu_sc as plsc

NC = 2
NS = 16


def _deg_call(ei3, n_pad):
    _, nrow, chunk = ei3.shape
    e = nrow * chunk
    epw = e // (NC * NS)
    nchunk = epw // chunk
    rpt = n_pad // NS

    mesh = plsc.VectorSubcoreMesh(core_axis_name="c", subcore_axis_name="s",
                                  num_cores=NC, num_subcores=NS)

    @functools.partial(
        pl.kernel,
        out_type=jax.ShapeDtypeStruct((NC, n_pad), jnp.float32),
        mesh=mesh,
        compiler_params=pltpu.CompilerParams(use_tc_tiling_on_sc=False),
        scratch_types=[
            pltpu.VMEM((nchunk, chunk), jnp.int32),
            pltpu.VMEM((chunk,), jnp.float32),
            pltpu.VMEM(((rpt + 15) // 16 * 16,), jnp.float32),
            pltpu.VMEM_SHARED((n_pad,), jnp.float32),
        ],
    )
    def degk(ei_hbm, out_hbm, dst_all, ones_v, stage_v, deg_sh):
        c = lax.axis_index("c")
        s = lax.axis_index("s")
        wid = s * NC + c

        pltpu.sync_copy(ei_hbm.at[1, pl.ds(wid * nchunk, nchunk), :], dst_all)

        def fill(i, _):
            ones_v[pl.ds(i * 16, 16)] = jnp.ones((16,), jnp.float32)
            return 0

        lax.fori_loop(0, chunk // 16, fill, 0)

        def fillz(i, _):
            stage_v[pl.ds(i * 16, 16)] = jnp.zeros((16,), jnp.float32)
            return 0

        lax.fori_loop(0, stage_v.shape[0] // 16, fillz, 0)
        pltpu.sync_copy(stage_v.at[pl.ds(0, rpt)],
                        deg_sh.at[pl.ds(s * rpt, rpt)])
        plsc.subcore_barrier()

        def body(k, _):
            pltpu.sync_copy(ones_v, deg_sh.at[dst_all.at[k]], add=True)
            return 0

        lax.fori_loop(0, nchunk, body, 0)
        plsc.subcore_barrier()
        pltpu.sync_copy(deg_sh.at[pl.ds(s * rpt, rpt)],
                        stage_v.at[pl.ds(0, rpt)])
        pltpu.sync_copy(stage_v.at[pl.ds(0, rpt)],
                        out_hbm.at[c, pl.ds(s * rpt, rpt)])

    return degk(ei3)


def _msgpass_call(y, ei3, n_pad):
    n, h = y.shape
    _, nrow, chunk = ei3.shape
    e = nrow * chunk
    epw = e // (NC * NS)
    nchunk = epw // chunk
    npair = (nchunk - 1) // 2
    rpt = n_pad // NS

    mesh = plsc.VectorSubcoreMesh(core_axis_name="c", subcore_axis_name="s",
                                  num_cores=NC, num_subcores=NS)

    @functools.partial(
        pl.kernel,
        out_type=[jax.ShapeDtypeStruct((n_pad, h), jnp.float32),
                  jax.ShapeDtypeStruct((n_pad, h), jnp.float32)],
        mesh=mesh,
        compiler_params=pltpu.CompilerParams(use_tc_tiling_on_sc=False),
        scratch_types=[
            pltpu.VMEM((nchunk, chunk), jnp.int32),
            pltpu.VMEM((nchunk, chunk), jnp.int32),
            pltpu.VMEM((chunk, h), jnp.float32),
            pltpu.VMEM((chunk, h), jnp.float32),
            pltpu.VMEM_SHARED((n_pad, h), jnp.float32),
            pltpu.SemaphoreType.DMA,
            pltpu.SemaphoreType.DMA,
        ],
    )
    def msgk(y_hbm, ei_hbm, out0_hbm, out1_hbm,
             src_all, dst_all, rows_a, rows_b,
             acc_sh, sem_a, sem_b):
        c = lax.axis_index("c")
        s = lax.axis_index("s")
        wid = s * NC + c

        pltpu.sync_copy(ei_hbm.at[0, pl.ds(wid * nchunk, nchunk), :], src_all)
        pltpu.sync_copy(ei_hbm.at[1, pl.ds(wid * nchunk, nchunk), :], dst_all)

        def fz(r, _):
            for j in range(h // 16):
                rows_a[r, pl.ds(j * 16, 16)] = jnp.zeros((16,), jnp.float32)
            return 0

        lax.fori_loop(0, chunk, fz, 0)
        left = rpt
        r0 = s * rpt
        while left > 0:
            m = min(left, chunk)
            pltpu.sync_copy(rows_a.at[pl.ds(0, m)],
                            acc_sh.at[pl.ds(r0, m)])
            r0 += m
            left -= m
        plsc.subcore_barrier()

        def start(k, rows, sem):
            pltpu.async_copy(y_hbm.at[src_all.at[k]], rows, sem)

        def wait_scatter(k, rows, sem):
            pltpu.make_async_copy(y_hbm.at[src_all.at[k]], rows, sem).wait()
            pltpu.sync_copy(rows, acc_sh.at[dst_all.at[k]], add=True)

        start(0, rows_a, sem_a)

        def body(j, _):
            start(2 * j + 1, rows_b, sem_b)
            wait_scatter(2 * j, rows_a, sem_a)
            start(2 * j + 2, rows_a, sem_a)
            wait_scatter(2 * j + 1, rows_b, sem_b)
            return 0

        lax.fori_loop(0, npair, body, 0)
        wait_scatter(nchunk - 1, rows_a, sem_a)

        plsc.subcore_barrier()
        left = rpt
        r0 = s * rpt
        while left > 0:
            m = min(left, chunk)
            pltpu.sync_copy(acc_sh.at[pl.ds(r0, m)],
                            rows_a.at[pl.ds(0, m)])

            @pl.when(c == 0)
            def _():
                pltpu.sync_copy(rows_a.at[pl.ds(0, m)],
                                out0_hbm.at[pl.ds(r0, m)])

            @pl.when(c == 1)
            def _():
                pltpu.sync_copy(rows_a.at[pl.ds(0, m)],
                                out1_hbm.at[pl.ds(r0, m)])

            r0 += m
            left -= m

    return msgk(y, ei3)


def _tc_stage1(x, W1, deg_t):
    n, d_in = x.shape
    h1 = W1.shape[1]
    bm = 2000
    g = n // bm

    def body(x_ref, w_ref, d_ref, y_ref, dis_ref):
        d = d_ref[...]
        dis = lax.rsqrt(d[:, 0:1] + d[:, 1:2] + 1.0)
        xw = jnp.dot(x_ref[...], w_ref[...],
                     preferred_element_type=jnp.float32)
        y_ref[...] = xw * dis
        dis_ref[...] = dis

    return pl.pallas_call(
        body,
        grid=(g,),
        in_specs=[
            pl.BlockSpec((bm, d_in), lambda i: (i, 0)),
            pl.BlockSpec((d_in, h1), lambda i: (0, 0)),
            pl.BlockSpec((bm, 2), lambda i: (i, 0)),
        ],
        out_specs=[
            pl.BlockSpec((bm, h1), lambda i: (i, 0)),
            pl.BlockSpec((bm, 1), lambda i: (i, 0)),
        ],
        out_shape=[
            jax.ShapeDtypeStruct((n, h1), jnp.float32),
            jax.ShapeDtypeStruct((n, 1), jnp.float32),
        ],
    )(x, W1, deg_t)


def _tc_stage2(p0, p1, y1, dis, b1, W2):
    n, h1 = y1.shape
    h2 = W2.shape[1]
    bm = 2000
    g = n // bm

    def body(p0_ref, p1_ref, y_ref, dis_ref, b_ref, w_ref, y2_ref):
        dis = dis_ref[...]
        hh = dis * (p0_ref[...] + p1_ref[...] + y_ref[...]) + b_ref[...]
        hh = jnp.maximum(hh, 0.0)
        y2_ref[...] = dis * jnp.dot(hh, w_ref[...],
                                    preferred_element_type=jnp.float32)

    return pl.pallas_call(
        body,
        grid=(g,),
        in_specs=[
            pl.BlockSpec((bm, h1), lambda i: (i, 0)),
            pl.BlockSpec((bm, h1), lambda i: (i, 0)),
            pl.BlockSpec((bm, h1), lambda i: (i, 0)),
            pl.BlockSpec((bm, 1), lambda i: (i, 0)),
            pl.BlockSpec((1, h1), lambda i: (0, 0)),
            pl.BlockSpec((h1, h2), lambda i: (0, 0)),
        ],
        out_specs=pl.BlockSpec((bm, h2), lambda i: (i, 0)),
        out_shape=jax.ShapeDtypeStruct((n, h2), jnp.float32),
    )(p0, p1, y1, dis, b1, W2)


def _tc_stage3(q0, q1, y2, dis, b2, Wl1, bl1, Wl2, bl2):
    n, h2 = y2.shape
    l2 = Wl2.shape[1]

    def body(q0_ref, q1_ref, y_ref, dis_ref, b2_ref, w1_ref, c1_ref,
             w2_ref, c2_ref, out_ref):
        hh = dis_ref[...] * (q0_ref[...] + q1_ref[...] + y_ref[...]) \
            + b2_ref[...]
        t = jnp.dot(hh, w1_ref[...],
                    preferred_element_type=jnp.float32) + c1_ref[...]
        lg = jnp.dot(t, w2_ref[...],
                     preferred_element_type=jnp.float32) + c2_ref[...]
        m = jnp.max(lg, axis=0, keepdims=True)
        ex = jnp.exp(lg - m)
        sm = jnp.sum(ex, axis=0, keepdims=True)
        out_ref[...] = ex / sm

    h1 = q0.shape[1]
    l1 = Wl1.shape[1]
    return pl.pallas_call(
        body,
        grid=(1,),
        in_specs=[
            pl.BlockSpec((n, h1), lambda i: (0, 0)),
            pl.BlockSpec((n, h1), lambda i: (0, 0)),
            pl.BlockSpec((n, h2), lambda i: (0, 0)),
            pl.BlockSpec((n, 1), lambda i: (0, 0)),
            pl.BlockSpec((1, h2), lambda i: (0, 0)),
            pl.BlockSpec((h2, l1), lambda i: (0, 0)),
            pl.BlockSpec((1, l1), lambda i: (0, 0)),
            pl.BlockSpec((l1, l2), lambda i: (0, 0)),
            pl.BlockSpec((1, l2), lambda i: (0, 0)),
        ],
        out_specs=pl.BlockSpec((n, l2), lambda i: (0, 0)),
        out_shape=jax.ShapeDtypeStruct((n, l2), jnp.float32),
    )(q0, q1, y2, dis, b2, Wl1, bl1, Wl2, bl2)


def kernel(x, edge_index, W1, b1, W2, b2, Wl1, bl1, Wl2, bl2):
    n = x.shape[0]
    h1 = W1.shape[1]
    n_pad = ((n + 16 * 8 - 1) // (16 * 8)) * (16 * 8)

    chunk = 400
    ei3 = edge_index.astype(jnp.int32).reshape(2, -1, chunk)

    deg_p = _deg_call(ei3, n_pad)
    deg_t = jnp.transpose(deg_p)

    y1, dis = _tc_stage1(x, W1, deg_t)

    p0, p1 = _msgpass_call(y1, ei3, n_pad)
    y2 = _tc_stage2(p0, p1, y1, dis, b1.reshape(1, -1), W2)

    q0, q1 = _msgpass_call(y2, ei3, n_pad)
    return _tc_stage3(q0, q1, y2, dis, b2.reshape(1, -1),
                      Wl1, bl1.reshape(1, -1), Wl2, bl2.reshape(1, -1))
